# baseline (speedup 1.0000x reference)
import dataclasses

import jax
import jax.numpy as jnp
from jax import lax
from jax.experimental import pallas as pl
from jax.experimental.pallas import tpu as pltpu
from jax.experimental.pallas import tpu_sc as plsc

_INV_CELL = 64.0
_C = 2304
_L = 16
_NMAIN = 999936
_NTAIL = 64
_B = 8192


def _frac(r):
    return r - r.astype(jnp.int32).astype(jnp.float32)


def _sf_planes(px, py, pz):
    fx = _frac(px * _INV_CELL)
    fy = _frac(py * _INV_CELL)
    fz = _frac(pz * _INV_CELL)
    bx = (1.0 - fx, fx)
    by = (1.0 - fy, fy)
    bz = (1.0 - fz, fz)
    bxy = {(i, j): bx[i] * by[j] for i in (0, 1) for j in (0, 1)}
    out = []
    for w in range(8):
        i, j, k = (w >> 2) & 1, (w >> 1) & 1, w & 1
        out.append(bxy[(i, j)] * bz[k])
    return out


def _sc_block_body(pos_vmem, sf_vmem):
    @pl.loop(0, _C, step=_L)
    def _(g):
        s = pl.ds(g, _L)
        sf = _sf_planes(pos_vmem[0, s], pos_vmem[1, s], pos_vmem[2, s])
        for w in range(8):
            sf_vmem[w, s] = sf[w]


def _sc_shapef(position_t):
    n = position_t.shape[1]
    mesh = plsc.VectorSubcoreMesh(core_axis_name="core", subcore_axis_name="subcore")
    cp = pltpu.CompilerParams()
    if "needs_layout_passes" in pltpu.CompilerParams.__dataclass_fields__:
        cp = dataclasses.replace(cp, needs_layout_passes=False)

    @pl.kernel(
        out_type=jax.ShapeDtypeStruct((8, n), jnp.float32),
        mesh=mesh,
        compiler_params=cp,
        scratch_types=[
            pltpu.VMEM((3, _NTAIL), jnp.float32),
            pltpu.VMEM((8, _NTAIL), jnp.float32),
        ],
    )
    def run(pos_hbm, sf_hbm, tp_v, ts_v):
        pltpu.emit_pipeline(
            _sc_block_body,
            grid=(_NMAIN // _C,),
            in_specs=[pl.BlockSpec((3, _C), lambda i: (0, i))],
            out_specs=[pl.BlockSpec((8, _C), lambda i: (0, i))],
            core_axis_name=("core", "subcore"),
            dimension_semantics=(pltpu.PARALLEL,),
        )(pos_hbm, sf_hbm)

        wid = lax.axis_index("subcore") * 2 + lax.axis_index("core")

        @pl.when(wid == 0)
        def _():
            pltpu.sync_copy(pos_hbm.at[:, pl.ds(_NMAIN, _NTAIL)], tp_v)

            @pl.loop(0, _NTAIL, step=_L)
            def _(g):
                s = pl.ds(g, _L)
                sf = _sf_planes(tp_v[0, s], tp_v[1, s], tp_v[2, s])
                for w in range(8):
                    ts_v[w, s] = sf[w]

            pltpu.sync_copy(ts_v, sf_hbm.at[:, pl.ds(_NMAIN, _NTAIL)])

    return run(position_t)


def _tc_grad_body(pos_ref, gf_ref):
    fx = _frac(pos_ref[0:1, :] * _INV_CELL)
    fy = _frac(pos_ref[1:2, :] * _INV_CELL)
    fz = _frac(pos_ref[2:3, :] * _INV_CELL)
    ox = 1.0 - fx
    oy = 1.0 - fy
    oz = 1.0 - fz
    pdx = jnp.sign(fx) * _INV_CELL
    pdy = jnp.sign(fy) * _INV_CELL
    pdz = jnp.sign(fz) * _INV_CELL

    b = pos_ref.shape[1]
    wi = lax.broadcasted_iota(jnp.int32, (8, b), 0)
    im = ((wi >> 2) & 1) == 1
    jm = ((wi >> 1) & 1) == 1
    km = (wi & 1) == 1

    bx8 = jnp.where(im, fx, ox)
    by8 = jnp.where(jm, fy, oy)
    bz8 = jnp.where(km, fz, oz)
    dbx8 = jnp.where(im, pdx, -pdx)
    dby8 = jnp.where(jm, pdy, -pdy)
    dbz8 = jnp.where(km, pdz, -pdz)

    gf_ref[0:8, :] = dbx8 * (by8 * bz8)
    gf_ref[8:16, :] = dby8 * (bx8 * bz8)
    gf_ref[16:24, :] = dbz8 * (bx8 * by8)


def _tc_grad(position_t):
    n = position_t.shape[1]
    grid = (pl.cdiv(n, _B),)
    return pl.pallas_call(
        _tc_grad_body,
        grid=grid,
        in_specs=[pl.BlockSpec((3, _B), lambda i: (0, i))],
        out_specs=pl.BlockSpec((24, _B), lambda i: (0, i)),
        out_shape=jax.ShapeDtypeStruct((24, n), jnp.float32),
    )(position_t)


def kernel(position_stack):
    n = position_stack.shape[0]
    assert n == _NMAIN + _NTAIL
    pos_t = position_stack.T
    sf = _sc_shapef(pos_t)
    gf = _tc_grad(pos_t)
    return sf.T, gf.reshape(3, 8, n).transpose(2, 1, 0)
